# baseline (device time: 36178 ns/iter reference)
import jax
import jax.numpy as jnp
from jax import lax
from jax.experimental import pallas as pl
from jax.experimental.pallas import tpu as pltpu

N_DEV = 32
N_BLOCKS = 8


def kernel(x, W1, W2):
    m, k = x.shape
    _, h_dim = W1.shape
    out_n = W2.shape[1]
    chunk = m // N_DEV
    blk = m // N_BLOCKS
    cpb = blk // chunk

    def body(x_ref, w1_ref, w2_ref, out_ref, part_ref, rs_recv,
             rs_send_sems, rs_recv_sems, ag_send_sems, ag_recv_sems):
        my = lax.axis_index("i")

        barrier_sem = pltpu.get_barrier_semaphore()
        for s in range(N_DEV - 1):
            peer = lax.rem(my + 1 + s, N_DEV)
            pl.semaphore_signal(
                barrier_sem, inc=1,
                device_id=(peer,), device_id_type=pl.DeviceIdType.MESH,
            )

        w1b = w1_ref[...].astype(jnp.bfloat16)
        w2b = w2_ref[...].astype(jnp.bfloat16)

        for b in range(N_BLOCKS):
            xb = x_ref[pl.ds(b * blk, blk), :].astype(jnp.bfloat16)
            h = jnp.dot(xb, w1b, preferred_element_type=jnp.float32)
            hb = jnp.maximum(h, 0.0).astype(jnp.bfloat16)
            pb = jnp.dot(hb, w2b, preferred_element_type=jnp.float32)
            part_ref[pl.ds(b * blk, blk), :] = pb.astype(jnp.bfloat16)
            if b == 0:
                pl.semaphore_wait(barrier_sem, N_DEV - 1)
            for i in range(cpb):
                c = b * cpb + i

                @pl.when(c != my)
                def _():
                    slot = lax.rem(my - c - 1 + 2 * N_DEV, N_DEV)
                    rdma = pltpu.make_async_remote_copy(
                        src_ref=part_ref.at[pl.ds(c * chunk, chunk), :],
                        dst_ref=rs_recv.at[slot],
                        send_sem=rs_send_sems.at[c],
                        recv_sem=rs_recv_sems.at[slot],
                        device_id=(jnp.int32(c),),
                        device_id_type=pl.DeviceIdType.MESH,
                    )
                    rdma.start()

        for s in range(N_DEV - 1):
            pltpu.make_async_remote_copy(
                src_ref=rs_recv.at[s],
                dst_ref=rs_recv.at[s],
                send_sem=rs_send_sems.at[s],
                recv_sem=rs_recv_sems.at[s],
                device_id=(my,),
                device_id_type=pl.DeviceIdType.MESH,
            ).wait_recv()

        own = part_ref[pl.ds(my * chunk, chunk), :].astype(jnp.float32)
        total = own + jnp.sum(rs_recv[...].astype(jnp.float32), axis=0)
        out_ref[pl.ds(my * chunk, chunk), :] = total.astype(jnp.bfloat16)

        ag = []
        for s in range(N_DEV - 1):
            j = lax.rem(my - 1 - s + 2 * N_DEV, N_DEV)
            rdma = pltpu.make_async_remote_copy(
                src_ref=out_ref.at[pl.ds(my * chunk, chunk), :],
                dst_ref=out_ref.at[pl.ds(my * chunk, chunk), :],
                send_sem=ag_send_sems.at[s],
                recv_sem=ag_recv_sems.at[s],
                device_id=(j,),
                device_id_type=pl.DeviceIdType.MESH,
            )
            rdma.start()
            ag.append(rdma)

        for c in range(N_DEV):

            @pl.when(c != my)
            def _():
                pltpu.make_async_remote_copy(
                    src_ref=part_ref.at[pl.ds(0, chunk), :],
                    dst_ref=rs_recv.at[0],
                    send_sem=rs_send_sems.at[c],
                    recv_sem=rs_recv_sems.at[0],
                    device_id=(my,),
                    device_id_type=pl.DeviceIdType.MESH,
                ).wait_send()

        for rdma in ag:
            rdma.wait_send()
        for rdma in ag:
            rdma.wait_recv()

    return pl.pallas_call(
        body,
        out_shape=jax.ShapeDtypeStruct((m, out_n), jnp.bfloat16),
        in_specs=[
            pl.BlockSpec(memory_space=pltpu.VMEM),
            pl.BlockSpec(memory_space=pltpu.VMEM),
            pl.BlockSpec(memory_space=pltpu.VMEM),
        ],
        out_specs=pl.BlockSpec(memory_space=pltpu.VMEM),
        scratch_shapes=[
            pltpu.VMEM((m, out_n), jnp.bfloat16),
            pltpu.VMEM((N_DEV - 1, chunk, out_n), jnp.bfloat16),
            pltpu.SemaphoreType.DMA((N_DEV,)),
            pltpu.SemaphoreType.DMA((N_DEV - 1,)),
            pltpu.SemaphoreType.DMA((N_DEV - 1,)),
            pltpu.SemaphoreType.DMA((N_DEV - 1,)),
        ],
        compiler_params=pltpu.CompilerParams(collective_id=0),
    )(x, W1, W2)


# device time: 32237 ns/iter; 1.1223x vs baseline; 1.1223x over previous
import jax
import jax.numpy as jnp
from jax import lax
from jax.experimental import pallas as pl
from jax.experimental.pallas import tpu as pltpu

N_DEV = 32
PLANE = 8
N_PLANES = 4


def kernel(x, W1, W2):
    m, k = x.shape
    _, h_dim = W1.shape
    out_n = W2.shape[1]
    blk = m // PLANE

    def body(x_ref, w1_ref, w2_ref, out_ref, part_ref, b_buf,
             a_recv, b_recv,
             a_send_sems, a_recv_sems,
             b_send_sems, b_recv_sems,
             c_send_sems, c_recv_sems):
        my = lax.axis_index("i")
        p = my // PLANE
        q = lax.rem(my, PLANE)

        barrier_sem = pltpu.get_barrier_semaphore()
        for s in range(PLANE - 1):
            peer = p * PLANE + lax.rem(q + 1 + s, PLANE)
            pl.semaphore_signal(
                barrier_sem, inc=1,
                device_id=(peer,), device_id_type=pl.DeviceIdType.MESH,
            )
        for u in range(N_PLANES - 1):
            peer = lax.rem(p + 1 + u, N_PLANES) * PLANE + q
            pl.semaphore_signal(
                barrier_sem, inc=1,
                device_id=(peer,), device_id_type=pl.DeviceIdType.MESH,
            )

        xb = x_ref[...].astype(jnp.bfloat16)
        w1b = w1_ref[...].astype(jnp.bfloat16)
        h = jnp.dot(xb, w1b, preferred_element_type=jnp.float32)
        hb = jnp.maximum(h, 0.0).astype(jnp.bfloat16)
        w2b = w2_ref[...].astype(jnp.bfloat16)
        part_ref[...] = jnp.dot(hb, w2b, preferred_element_type=jnp.float32
                                ).astype(jnp.bfloat16)

        pl.semaphore_wait(barrier_sem, (PLANE - 1) + (N_PLANES - 1))

        a = []
        for s in range(PLANE - 1):
            q_dst = lax.rem(q - 1 - s + 2 * PLANE, PLANE)
            rdma = pltpu.make_async_remote_copy(
                src_ref=part_ref.at[pl.ds(q_dst * blk, blk), :],
                dst_ref=a_recv.at[s],
                send_sem=a_send_sems.at[s],
                recv_sem=a_recv_sems.at[s],
                device_id=(p * PLANE + q_dst,),
                device_id_type=pl.DeviceIdType.MESH,
            )
            rdma.start()
            a.append(rdma)
        for rdma in a:
            rdma.wait_recv()

        own = part_ref[pl.ds(q * blk, blk), :].astype(jnp.float32)
        red = own + jnp.sum(a_recv[...].astype(jnp.float32), axis=0)
        b_buf[...] = red.astype(jnp.bfloat16)

        b = []
        for u in range(N_PLANES - 1):
            p_dst = lax.rem(p - 1 - u + 2 * N_PLANES, N_PLANES)
            rdma = pltpu.make_async_remote_copy(
                src_ref=b_buf,
                dst_ref=b_recv.at[u],
                send_sem=b_send_sems.at[u],
                recv_sem=b_recv_sems.at[u],
                device_id=(p_dst * PLANE + q,),
                device_id_type=pl.DeviceIdType.MESH,
            )
            rdma.start()
            b.append(rdma)
        for rdma in b:
            rdma.wait_recv()

        total = red + jnp.sum(b_recv[...].astype(jnp.float32), axis=0)
        out_ref[pl.ds(q * blk, blk), :] = total.astype(jnp.bfloat16)

        c = []
        for s in range(PLANE - 1):
            q_dst = lax.rem(q - 1 - s + 2 * PLANE, PLANE)
            rdma = pltpu.make_async_remote_copy(
                src_ref=out_ref.at[pl.ds(q * blk, blk), :],
                dst_ref=out_ref.at[pl.ds(q * blk, blk), :],
                send_sem=c_send_sems.at[s],
                recv_sem=c_recv_sems.at[s],
                device_id=(p * PLANE + q_dst,),
                device_id_type=pl.DeviceIdType.MESH,
            )
            rdma.start()
            c.append(rdma)

        for rdma in a:
            rdma.wait_send()
        for rdma in b:
            rdma.wait_send()
        for rdma in c:
            rdma.wait_send()
        for rdma in c:
            rdma.wait_recv()

    return pl.pallas_call(
        body,
        out_shape=jax.ShapeDtypeStruct((m, out_n), jnp.bfloat16),
        in_specs=[
            pl.BlockSpec(memory_space=pltpu.VMEM),
            pl.BlockSpec(memory_space=pltpu.VMEM),
            pl.BlockSpec(memory_space=pltpu.VMEM),
        ],
        out_specs=pl.BlockSpec(memory_space=pltpu.VMEM),
        scratch_shapes=[
            pltpu.VMEM((m, out_n), jnp.bfloat16),
            pltpu.VMEM((blk, out_n), jnp.bfloat16),
            pltpu.VMEM((PLANE - 1, blk, out_n), jnp.bfloat16),
            pltpu.VMEM((N_PLANES - 1, blk, out_n), jnp.bfloat16),
            pltpu.SemaphoreType.DMA((PLANE - 1,)),
            pltpu.SemaphoreType.DMA((PLANE - 1,)),
            pltpu.SemaphoreType.DMA((N_PLANES - 1,)),
            pltpu.SemaphoreType.DMA((N_PLANES - 1,)),
            pltpu.SemaphoreType.DMA((PLANE - 1,)),
            pltpu.SemaphoreType.DMA((PLANE - 1,)),
        ],
        compiler_params=pltpu.CompilerParams(collective_id=0),
    )(x, W1, W2)
